# baseline (speedup 1.0000x reference)
import jax
import jax.numpy as jnp
from jax import lax
from jax.experimental import pallas as pl
from jax.experimental.pallas import tpu as pltpu
from jax.experimental.pallas import tpu_sc as plsc

B = 16384
D = 32
H = 8
UPM = 16
MD = 128
NV = 1000000
NM = NV // UPM
NC = 2
NS = 16
NW = NC * NS
BPW = B // NW
SEG = 128
NSEG = BPW // SEG
TRN = 16384
ROWS_TC = 2048


def _proj_body(ut_ref, it_ref, w1u_ref, w1i_ref, ou_ref, oi_ref):
    pu = jnp.dot(w1u_ref[...], ut_ref[...], preferred_element_type=jnp.float32)
    pi = jnp.dot(w1i_ref[...], it_ref[...], preferred_element_type=jnp.float32)
    for c in range(TRN // 512):
        om = pl.ds(c * 32, 32)
        tu = pu[:, c * 512:(c + 1) * 512].T.reshape(32, UPM, H)
        ti = pi[:, c * 512:(c + 1) * 512].T.reshape(32, UPM, H)
        ou_ref[om, :] = jnp.concatenate([tu[:, b, :] for b in range(UPM)], axis=1)
        oi_ref[om, :] = jnp.concatenate([ti[:, b, :] for b in range(UPM)], axis=1)


def _gather_body(uidx_hbm, iidx_hbm, embu_hbm, embi_hbm, gu_hbm, gi_hbm,
                 uidx_v, iidx_v, urows_v, irows_v, usem, isem):
    c = lax.axis_index("c")
    s = lax.axis_index("s")
    wid = s * NC + c

    pltpu.sync_copy(uidx_hbm.at[wid], uidx_v)
    pltpu.sync_copy(iidx_hbm.at[wid], iidx_v)

    base = wid * BPW
    for g in range(NSEG):
        cu = pltpu.async_copy(embu_hbm.at[uidx_v.at[g]], urows_v, usem)
        ci = pltpu.async_copy(embi_hbm.at[iidx_v.at[g]], irows_v, isem)
        cu.wait()
        pltpu.sync_copy(urows_v, gu_hbm.at[pl.ds(base + g * SEG, SEG)])
        ci.wait()
        pltpu.sync_copy(irows_v, gi_hbm.at[pl.ds(base + g * SEG, SEG)])


def _mlp_body(uidx_ref, iidx_ref, gu_ref, gi_ref, b1_ref,
              w2t_ref, b2_ref, wat_ref, ba_ref, out_ref):
    usub = uidx_ref[...] & (UPM - 1)
    isub = iidx_ref[...] & (UPM - 1)
    rows = uidx_ref.shape[0]
    gu4 = gu_ref[...].reshape(rows, UPM, H)
    gi4 = gi_ref[...].reshape(rows, UPM, H)
    bids = lax.broadcasted_iota(jnp.int32, (1, UPM, 1), 1)
    pu = jnp.sum(jnp.where(usub[:, :, None] == bids, gu4, 0.0), axis=1)
    pi = jnp.sum(jnp.where(isub[:, :, None] == bids, gi4, 0.0), axis=1)
    h1 = jnp.maximum(pu + pi + b1_ref[...], 0.0)
    h2 = jnp.dot(h1, w2t_ref[...], preferred_element_type=jnp.float32) + b2_ref[...]
    h2 = jnp.maximum(h2, 0.0)
    logits = jnp.dot(h2, wat_ref[...], preferred_element_type=jnp.float32) + ba_ref[0, 0]
    out_ref[...] = 1.0 / (1.0 + jnp.exp(-logits))


def kernel(user_indices, item_indices, emb_user, emb_item, W1, b1, W2, b2, Wa, ba):
    nblk = pl.cdiv(NV, TRN)
    proj = pl.pallas_call(
        _proj_body,
        grid=(nblk,),
        in_specs=[
            pl.BlockSpec((D, TRN), lambda i: (0, i)),
            pl.BlockSpec((D, TRN), lambda i: (0, i)),
            pl.BlockSpec((H, D), lambda i: (0, 0)),
            pl.BlockSpec((H, D), lambda i: (0, 0)),
        ],
        out_specs=[
            pl.BlockSpec((TRN // UPM, MD), lambda i: (i, 0)),
            pl.BlockSpec((TRN // UPM, MD), lambda i: (i, 0)),
        ],
        out_shape=(jax.ShapeDtypeStruct((NM, MD), jnp.float32),
                   jax.ShapeDtypeStruct((NM, MD), jnp.float32)),
    )
    pu4, pi4 = proj(emb_user.T, emb_item.T, W1[:, :D], W1[:, D:])

    uq = (user_indices >> 4).reshape(NW, NSEG, SEG)
    iq = (item_indices >> 4).reshape(NW, NSEG, SEG)

    gather = pl.kernel(
        _gather_body,
        out_type=(jax.ShapeDtypeStruct((B, MD), jnp.float32),
                  jax.ShapeDtypeStruct((B, MD), jnp.float32)),
        mesh=plsc.VectorSubcoreMesh(core_axis_name="c", subcore_axis_name="s"),
        scratch_types=[
            pltpu.VMEM((NSEG, SEG), jnp.int32),
            pltpu.VMEM((NSEG, SEG), jnp.int32),
            pltpu.VMEM((SEG, MD), jnp.float32),
            pltpu.VMEM((SEG, MD), jnp.float32),
            pltpu.SemaphoreType.DMA,
            pltpu.SemaphoreType.DMA,
        ],
    )
    gu4, gi4 = gather(uq, iq, pu4, pi4)

    grid = (B // ROWS_TC,)
    full = lambda s: pl.BlockSpec(s, lambda i: (0, 0))
    out = pl.pallas_call(
        _mlp_body,
        grid=grid,
        in_specs=[
            pl.BlockSpec((ROWS_TC, 1), lambda i: (i, 0)),
            pl.BlockSpec((ROWS_TC, 1), lambda i: (i, 0)),
            pl.BlockSpec((ROWS_TC, MD), lambda i: (i, 0)),
            pl.BlockSpec((ROWS_TC, MD), lambda i: (i, 0)),
            full((1, 8)),
            full((8, 8)),
            full((1, 8)),
            full((8, 1)),
            full((1, 1)),
        ],
        out_specs=pl.BlockSpec((ROWS_TC, 1), lambda i: (i, 0)),
        out_shape=jax.ShapeDtypeStruct((B, 1), jnp.float32),
    )(user_indices.reshape(B, 1), item_indices.reshape(B, 1), gu4, gi4,
      b1.reshape(1, 8), W2.T, b2.reshape(1, 8), Wa.T, ba.reshape(1, 1))
    return out


# speedup vs baseline: 1.0836x; 1.0836x over previous
import jax
import jax.numpy as jnp
from jax import lax
from jax.experimental import pallas as pl
from jax.experimental.pallas import tpu as pltpu
from jax.experimental.pallas import tpu_sc as plsc

B = 16384
D = 32
MD = 128
NC = 2
NS = 16
NW = NC * NS
BPW = B // NW
SEG = 128
NSEG = BPW // SEG

ROWS_TC = 2048


def _gather_body(uidx_hbm, iidx_hbm, embu_hbm, embi_hbm, gu_hbm, gi_hbm,
                 uidx_v, iidx_v, urows_v, irows_v, usem, isem):
    c = lax.axis_index("c")
    s = lax.axis_index("s")
    wid = s * NC + c

    pltpu.sync_copy(uidx_hbm.at[wid], uidx_v)
    pltpu.sync_copy(iidx_hbm.at[wid], iidx_v)

    base = wid * BPW
    for g in range(NSEG):
        cu = pltpu.async_copy(embu_hbm.at[uidx_v.at[g]], urows_v, usem)
        ci = pltpu.async_copy(embi_hbm.at[iidx_v.at[g]], irows_v, isem)
        cu.wait()
        pltpu.sync_copy(urows_v, gu_hbm.at[pl.ds(base + g * SEG, SEG)])
        ci.wait()
        pltpu.sync_copy(irows_v, gi_hbm.at[pl.ds(base + g * SEG, SEG)])


TRN = 16384
NV = 1000000


def _tr_body(ut_ref, it_ref, ou_ref, oi_ref):
    for c in range(TRN // 128):
        sl = pl.ds(c * 128, 128)
        o4 = pl.ds(c * 32, 32)
        tu = ut_ref[:, sl].T.reshape(32, 4, D)
        ti = it_ref[:, sl].T.reshape(32, 4, D)
        ou_ref[o4, :] = jnp.concatenate([tu[:, a, :] for a in range(4)], axis=1)
        oi_ref[o4, :] = jnp.concatenate([ti[:, a, :] for a in range(4)], axis=1)


def _mlp_body(uidx_ref, iidx_ref, gu_ref, gi_ref, w1ut_ref, w1it_ref, b1_ref,
              w2t_ref, b2_ref, wat_ref, ba_ref, out_ref):
    usub = uidx_ref[...] & 3
    isub = iidx_ref[...] & 3
    gu4 = gu_ref[...]
    gi4 = gi_ref[...]
    gu = jnp.where(usub == 0, gu4[:, 0:D], gu4[:, D:2 * D])
    gi = jnp.where(isub == 0, gi4[:, 0:D], gi4[:, D:2 * D])
    for g in (2, 3):
        gu = jnp.where(usub == g, gu4[:, g * D:(g + 1) * D], gu)
        gi = jnp.where(isub == g, gi4[:, g * D:(g + 1) * D], gi)
    h1 = (jnp.dot(gu, w1ut_ref[...], preferred_element_type=jnp.float32)
          + jnp.dot(gi, w1it_ref[...], preferred_element_type=jnp.float32)
          + b1_ref[...])
    h1 = jnp.maximum(h1, 0.0)
    h2 = jnp.dot(h1, w2t_ref[...], preferred_element_type=jnp.float32) + b2_ref[...]
    h2 = jnp.maximum(h2, 0.0)
    logits = jnp.dot(h2, wat_ref[...], preferred_element_type=jnp.float32) + ba_ref[0, 0]
    out_ref[...] = 1.0 / (1.0 + jnp.exp(-logits))


def kernel(user_indices, item_indices, emb_user, emb_item, W1, b1, W2, b2, Wa, ba):
    tr = pl.pallas_call(
        _tr_body,
        grid=(pl.cdiv(NV, TRN),),
        in_specs=[
            pl.BlockSpec((D, TRN), lambda i: (0, i)),
            pl.BlockSpec((D, TRN), lambda i: (0, i)),
        ],
        out_specs=[
            pl.BlockSpec((TRN // 4, MD), lambda i: (i, 0)),
            pl.BlockSpec((TRN // 4, MD), lambda i: (i, 0)),
        ],
        out_shape=(jax.ShapeDtypeStruct((NV // 4, MD), jnp.float32),
                   jax.ShapeDtypeStruct((NV // 4, MD), jnp.float32)),
    )
    embu4, embi4 = tr(emb_user.T, emb_item.T)
    uq = (user_indices >> 2).reshape(NW, NSEG, SEG)
    iq = (item_indices >> 2).reshape(NW, NSEG, SEG)

    gather = pl.kernel(
        _gather_body,
        out_type=(jax.ShapeDtypeStruct((B, MD), jnp.float32),
                  jax.ShapeDtypeStruct((B, MD), jnp.float32)),
        mesh=plsc.VectorSubcoreMesh(core_axis_name="c", subcore_axis_name="s"),
        scratch_types=[
            pltpu.VMEM((NSEG, SEG), jnp.int32),
            pltpu.VMEM((NSEG, SEG), jnp.int32),
            pltpu.VMEM((SEG, MD), jnp.float32),
            pltpu.VMEM((SEG, MD), jnp.float32),
            pltpu.SemaphoreType.DMA,
            pltpu.SemaphoreType.DMA,
        ],
    )
    gu4, gi4 = gather(uq, iq, embu4, embi4)

    grid = (B // ROWS_TC,)
    full = lambda s: pl.BlockSpec(s, lambda i: (0, 0))
    out = pl.pallas_call(
        _mlp_body,
        grid=grid,
        in_specs=[
            pl.BlockSpec((ROWS_TC, 1), lambda i: (i, 0)),
            pl.BlockSpec((ROWS_TC, 1), lambda i: (i, 0)),
            pl.BlockSpec((ROWS_TC, MD), lambda i: (i, 0)),
            pl.BlockSpec((ROWS_TC, MD), lambda i: (i, 0)),
            full((D, 8)),
            full((D, 8)),
            full((1, 8)),
            full((8, 8)),
            full((1, 8)),
            full((8, 1)),
            full((1, 1)),
        ],
        out_specs=pl.BlockSpec((ROWS_TC, 1), lambda i: (i, 0)),
        out_shape=jax.ShapeDtypeStruct((B, 1), jnp.float32),
    )(user_indices.reshape(B, 1), item_indices.reshape(B, 1), gu4, gi4,
      W1[:, :D].T, W1[:, D:].T, b1.reshape(1, 8),
      W2.T, b2.reshape(1, 8), Wa.T, ba.reshape(1, 1))
    return out
